# baseline (speedup 1.0000x reference)
import functools

import jax
import jax.numpy as jnp
from jax import lax
from jax.experimental import pallas as pl
from jax.experimental.pallas import tpu as pltpu
from jax.experimental.pallas import tpu_sc as plsc

N_U = 20000
N_I = 30000
NN = 50000
D = 64
EDG = 800000
HALF = 32
NC = 2
NS = 16
CHUNK = 128
KB = 2
BLK = KB * CHUNK
G = 4
GB = G * KB
NG_SUB = 49
NB_SUB = NG_SUB * G
EPAD = NB_SUB * NS * BLK
ZCH = 200
TOT_ZCH = NN // ZCH

_mesh = plsc.VectorSubcoreMesh(core_axis_name="c", subcore_axis_name="s")


@functools.partial(
    pl.kernel,
    out_type=jax.ShapeDtypeStruct((NC, NN, HALF), jnp.float32),
    mesh=_mesh,
    scratch_types=[
        pltpu.VMEM((GB, 3, CHUNK), jnp.int32),
        pltpu.VMEM((GB, 3, CHUNK), jnp.int32),
        pltpu.VMEM((BLK, HALF), jnp.float32),
        pltpu.VMEM((BLK, HALF), jnp.float32),
        pltpu.VMEM_SHARED((NN, HALF), jnp.float32),
        pltpu.SemaphoreType.DMA,
        pltpu.SemaphoreType.DMA,
        pltpu.SemaphoreType.DMA,
        pltpu.SemaphoreType.DMA,
        pltpu.SemaphoreType.DMA,
    ],
    compiler_params=pltpu.CompilerParams(use_tc_tiling_on_sc=False,
                                         needs_layout_passes=False),
)
def _spmm(tbl, edat, out,
          ibA, ibB, rowsA, rowsB, acc,
          gA, gB, scA, scB, sidx):
    c = lax.axis_index("c")
    s = lax.axis_index("s")
    tblc = tbl.at[c]
    outc = out.at[c]
    dummy = tblc.at[pl.ds(0, BLK)]

    zero = jnp.zeros((16,), jnp.float32)

    def zfill(i, carry):
        rowsA[i, pl.ds(0, 16)] = zero
        rowsA[i, pl.ds(16, 16)] = zero
        return carry
    lax.fori_loop(0, ZCH, zfill, 0)

    nzc = jnp.where(s < TOT_ZCH % NS, TOT_ZCH // NS + 1, TOT_ZCH // NS)

    def zbody(j, carry):
        pltpu.sync_copy(rowsA.at[pl.ds(0, ZCH)],
                        acc.at[pl.ds((s + j * NS) * ZCH, ZCH)])
        return carry
    lax.fori_loop(0, nzc, zbody, 0)
    plsc.subcore_barrier()

    rowsP = (rowsA, rowsB)
    gP = (gA, gB)
    scP = (scA, scB)
    ibP = (ibA, ibB)

    def fire_gathers(ibuf, krow, rows_buf, gsem):
        for k in range(KB):
            pltpu.async_copy(tblc.at[ibuf.at[krow + k, 0]],
                             rows_buf.at[pl.ds(k * CHUNK, CHUNK)], gsem)

    def fire_scatters(ibuf, krow, rows_buf, scsem):
        for k in range(KB):
            pltpu.async_copy(rows_buf.at[pl.ds(k * CHUNK, CHUNK)],
                             acc.at[ibuf.at[krow + k, 1]], scsem, add=True)

    def mul_block(ibuf, krow, rows_buf):
        def mbody(q, carry):
            kr = krow + q // 8
            col = (q % 8) * 16
            wv = plsc.bitcast(ibuf[kr, 2, pl.ds(col, 16)], jnp.float32)
            for l in range(16):
                e = q * 16 + l
                rows_buf[e, pl.ds(0, 16)] = rows_buf[e, pl.ds(0, 16)] * wv[l]
                rows_buf[e, pl.ds(16, 16)] = rows_buf[e, pl.ds(16, 16)] * wv[l]
            return carry
        lax.fori_loop(0, BLK // 16, mbody, 0)

    def do_group(j, ig, first):
        ib, ibn = ibP[ig], ibP[1 - ig]
        jn = jnp.minimum(j + 1, NG_SUB - 1)
        nref = edat.at[pl.ds((s + jn * NS) * GB, GB)]
        for b in range(G):
            rp = b % 2
            rows_c, g_c, sc_c = rowsP[rp], gP[rp], scP[rp]
            rows_n, g_n, sc_n = rowsP[1 - rp], gP[1 - rp], scP[1 - rp]
            if not (first and b == 0):
                pltpu.make_async_copy(dummy, rows_n, sc_n).wait()
            if b == 0:
                pltpu.async_copy(nref, ibn, sidx)
            if b == G - 1:
                pltpu.make_async_copy(nref, ibn, sidx).wait()
                nib, nkrow = ibn, 0
            else:
                nib, nkrow = ib, (b + 1) * KB
            fire_gathers(nib, nkrow, rows_n, g_n)
            pltpu.make_async_copy(dummy, rows_c, g_c).wait()
            mul_block(ib, b * KB, rows_c)
            fire_scatters(ib, b * KB, rows_c, sc_c)

    pref0 = edat.at[pl.ds(s * GB, GB)]
    pltpu.async_copy(pref0, ibA, sidx)
    pltpu.make_async_copy(pref0, ibA, sidx).wait()
    fire_gathers(ibA, 0, rowsA, gA)
    do_group(0, 0, first=True)

    def pairbody(j2, carry):
        do_group(2 * j2 + 1, 1, first=False)
        do_group(2 * j2 + 2, 0, first=False)
        return carry
    lax.fori_loop(0, (NG_SUB - 1) // 2, pairbody, 0)

    pltpu.make_async_copy(dummy, rowsB, scB).wait()
    pltpu.make_async_copy(dummy, rowsA, gA).wait()
    plsc.subcore_barrier()

    def obody(j, carry):
        r0 = (s + j * NS) * ZCH
        pltpu.sync_copy(acc.at[pl.ds(r0, ZCH)], rowsA.at[pl.ds(0, ZCH)])
        pltpu.sync_copy(rowsA.at[pl.ds(0, ZCH)], outc.at[pl.ds(r0, ZCH)])
        return carry
    lax.fori_loop(0, nzc, obody, 0)


BN = 1000


def _dense_body(f_ref, x_ref, fw_ref, fb_ref, bw_ref, bb_ref, y_ref, yn_ref):
    f = f_ref[...]
    x = x_ref[...]
    F = jnp.concatenate([f[0], f[1]], axis=1)
    X = jnp.concatenate([x[0], x[1]], axis=1)
    S = F + X
    fc = lax.dot_general(S, fw_ref[...], (((1,), (1,)), ((), ())),
                         preferred_element_type=jnp.float32) + 2.0 * fb_ref[...]
    fc = jnp.where(fc >= 0, fc, 0.01 * fc)
    Bm = F * X
    bk = lax.dot_general(Bm, bw_ref[...], (((1,), (1,)), ((), ())),
                         preferred_element_type=jnp.float32) + bb_ref[...]
    bk = jnp.where(bk >= 0, bk, 0.01 * bk)
    Y = fc + bk
    y_ref[0] = Y[:, :HALF]
    y_ref[1] = Y[:, HALF:]
    nrm = jnp.sqrt(jnp.sum(Y * Y, axis=1, keepdims=True))
    yn_ref[...] = Y / jnp.maximum(nrm, 1e-12)


_dense = pl.pallas_call(
    _dense_body,
    grid=(NN // BN,),
    in_specs=[
        pl.BlockSpec((NC, BN, HALF), lambda i: (0, i, 0)),
        pl.BlockSpec((NC, BN, HALF), lambda i: (0, i, 0)),
        pl.BlockSpec((D, D), lambda i: (0, 0)),
        pl.BlockSpec((1, D), lambda i: (0, 0)),
        pl.BlockSpec((D, D), lambda i: (0, 0)),
        pl.BlockSpec((1, D), lambda i: (0, 0)),
    ],
    out_specs=[
        pl.BlockSpec((NC, BN, HALF), lambda i: (0, i, 0)),
        pl.BlockSpec((BN, D), lambda i: (i, 0)),
    ],
    out_shape=[
        jax.ShapeDtypeStruct((NC, NN, HALF), jnp.float32),
        jax.ShapeDtypeStruct((NN, D), jnp.float32),
    ],
)


def kernel(edge_index, edge_weight, user_emb, item_emb,
           fw0, fb0, fw1, fb1, fw2, fb2,
           bw0, bb0, bw1, bb1, bw2, bb2):
    pad = EPAD - EDG
    src = jnp.concatenate([edge_index[0].astype(jnp.int32),
                           jnp.zeros((pad,), jnp.int32)])
    dst = jnp.concatenate([edge_index[1].astype(jnp.int32),
                           jnp.zeros((pad,), jnp.int32)])
    w = jnp.concatenate([edge_weight.astype(jnp.float32),
                         jnp.zeros((pad,), jnp.float32)])
    wbits = lax.bitcast_convert_type(w, jnp.int32)
    edat = jnp.stack([src.reshape(-1, CHUNK), dst.reshape(-1, CHUNK),
                      wbits.reshape(-1, CHUNK)], axis=1)
    E0 = jnp.concatenate([user_emb, item_emb], axis=0)
    x3 = jnp.stack([E0[:, :HALF], E0[:, HALF:]], axis=0)
    layers = [(fw0, fb0, bw0, bb0), (fw1, fb1, bw1, bb1), (fw2, fb2, bw2, bb2)]
    outs = [E0]
    for (fw, fb, bw, bb) in layers:
        front3 = _spmm(x3, edat)
        y3, yn = _dense(front3, x3,
                        fw, fb.reshape(1, D), bw, bb.reshape(1, D))
        x3 = y3
        outs.append(yn)
    all_emb = jnp.concatenate(outs, axis=1)
    return all_emb[:N_U], all_emb[N_U:]


# speedup vs baseline: 1.0728x; 1.0728x over previous
import functools

import jax
import jax.numpy as jnp
from jax import lax
from jax.experimental import pallas as pl
from jax.experimental.pallas import tpu as pltpu
from jax.experimental.pallas import tpu_sc as plsc

N_U = 20000
N_I = 30000
NN = 50000
D = 64
EDG = 800000
HALF = 32
NC = 2
NS = 16
CHUNK = 128
KB = 2
BLK = KB * CHUNK
G = 4
GB = G * KB
NG_SUB = 49
NB_SUB = NG_SUB * G
EPAD = NB_SUB * NS * BLK
ZCH = 200
TOT_ZCH = NN // ZCH

_mesh = plsc.VectorSubcoreMesh(core_axis_name="c", subcore_axis_name="s")


@functools.partial(
    pl.kernel,
    out_type=jax.ShapeDtypeStruct((NC, NN, HALF), jnp.float32),
    mesh=_mesh,
    scratch_types=[
        pltpu.VMEM((3, GB, CHUNK), jnp.int32),
        pltpu.VMEM((3, GB, CHUNK), jnp.int32),
        pltpu.VMEM((BLK, HALF), jnp.float32),
        pltpu.VMEM((BLK, HALF), jnp.float32),
        pltpu.VMEM_SHARED((NN, HALF), jnp.float32),
        pltpu.SemaphoreType.DMA,
        pltpu.SemaphoreType.DMA,
        pltpu.SemaphoreType.DMA,
        pltpu.SemaphoreType.DMA,
        pltpu.SemaphoreType.DMA,
    ],
    compiler_params=pltpu.CompilerParams(use_tc_tiling_on_sc=False,
                                         needs_layout_passes=False),
)
def _spmm(tbl, edat, out,
          ibA, ibB, rowsA, rowsB, acc,
          gA, gB, scA, scB, sidx):
    c = lax.axis_index("c")
    s = lax.axis_index("s")
    tblc = tbl.at[c]
    outc = out.at[c]
    dummy = tblc.at[pl.ds(0, BLK)]

    zero = jnp.zeros((16,), jnp.float32)

    def zfill(i, carry):
        rowsA[i, pl.ds(0, 16)] = zero
        rowsA[i, pl.ds(16, 16)] = zero
        return carry
    lax.fori_loop(0, ZCH, zfill, 0)

    nzc = jnp.where(s < TOT_ZCH % NS, TOT_ZCH // NS + 1, TOT_ZCH // NS)

    def zbody(j, carry):
        pltpu.sync_copy(rowsA.at[pl.ds(0, ZCH)],
                        acc.at[pl.ds((s + j * NS) * ZCH, ZCH)])
        return carry
    lax.fori_loop(0, nzc, zbody, 0)
    plsc.subcore_barrier()

    rowsP = (rowsA, rowsB)
    gP = (gA, gB)
    scP = (scA, scB)
    ibP = (ibA, ibB)

    def fire_gathers(ibuf, krow, rows_buf, gsem):
        for k in range(KB):
            pltpu.async_copy(tblc.at[ibuf.at[0, krow + k]],
                             rows_buf.at[pl.ds(k * CHUNK, CHUNK)], gsem)

    def fire_scatters(ibuf, krow, rows_buf, scsem):
        for k in range(KB):
            pltpu.async_copy(rows_buf.at[pl.ds(k * CHUNK, CHUNK)],
                             acc.at[ibuf.at[1, krow + k]], scsem, add=True)

    def mul_block(ibuf, krow, rows_buf):
        def mbody(q, carry):
            kr = krow + q // 8
            col = (q % 8) * 16
            wv = plsc.bitcast(ibuf[2, kr, pl.ds(col, 16)], jnp.float32)
            for l in range(16):
                e = q * 16 + l
                rows_buf[e, pl.ds(0, 16)] = rows_buf[e, pl.ds(0, 16)] * wv[l]
                rows_buf[e, pl.ds(16, 16)] = rows_buf[e, pl.ds(16, 16)] * wv[l]
            return carry
        lax.fori_loop(0, BLK // 16, mbody, 0)

    def fire_group_load(r0, ibuf):
        for i in range(3):
            pltpu.async_copy(edat.at[i, pl.ds(r0, GB)], ibuf.at[i], sidx)

    def drain_group_load(r0, ibuf):
        for i in range(3):
            pltpu.make_async_copy(edat.at[i, pl.ds(r0, GB)],
                                  ibuf.at[i], sidx).wait()

    def do_group(j, ig, first):
        ib, ibn = ibP[ig], ibP[1 - ig]
        jn = jnp.minimum(j + 1, NG_SUB - 1)
        nr0 = (s + jn * NS) * GB
        for b in range(G):
            rp = b % 2
            rows_c, g_c, sc_c = rowsP[rp], gP[rp], scP[rp]
            rows_n, g_n, sc_n = rowsP[1 - rp], gP[1 - rp], scP[1 - rp]
            if not (first and b == 0):
                pltpu.make_async_copy(dummy, rows_n, sc_n).wait()
            if b == 0:
                fire_group_load(nr0, ibn)
            if b == G - 1:
                drain_group_load(nr0, ibn)
                nib, nkrow = ibn, 0
            else:
                nib, nkrow = ib, (b + 1) * KB
            fire_gathers(nib, nkrow, rows_n, g_n)
            pltpu.make_async_copy(dummy, rows_c, g_c).wait()
            mul_block(ib, b * KB, rows_c)
            fire_scatters(ib, b * KB, rows_c, sc_c)

    fire_group_load(s * GB, ibA)
    drain_group_load(s * GB, ibA)
    fire_gathers(ibA, 0, rowsA, gA)
    do_group(0, 0, first=True)

    def pairbody(j2, carry):
        do_group(2 * j2 + 1, 1, first=False)
        do_group(2 * j2 + 2, 0, first=False)
        return carry
    lax.fori_loop(0, (NG_SUB - 1) // 2, pairbody, 0)

    pltpu.make_async_copy(dummy, rowsB, scB).wait()
    pltpu.make_async_copy(dummy, rowsA, gA).wait()
    plsc.subcore_barrier()

    def obody(j, carry):
        r0 = (s + j * NS) * ZCH
        pltpu.sync_copy(acc.at[pl.ds(r0, ZCH)], rowsA.at[pl.ds(0, ZCH)])
        pltpu.sync_copy(rowsA.at[pl.ds(0, ZCH)], outc.at[pl.ds(r0, ZCH)])
        return carry
    lax.fori_loop(0, nzc, obody, 0)


BN = 2000


def _dense_body(f_ref, x_ref, fw_ref, fb_ref, bw_ref, bb_ref, y_ref, yn_ref):
    f = f_ref[...]
    x = x_ref[...]
    F = jnp.concatenate([f[0], f[1]], axis=1)
    X = jnp.concatenate([x[0], x[1]], axis=1)
    S = F + X
    fc = lax.dot_general(S, fw_ref[...], (((1,), (1,)), ((), ())),
                         preferred_element_type=jnp.float32) + 2.0 * fb_ref[...]
    fc = jnp.where(fc >= 0, fc, 0.01 * fc)
    Bm = F * X
    bk = lax.dot_general(Bm, bw_ref[...], (((1,), (1,)), ((), ())),
                         preferred_element_type=jnp.float32) + bb_ref[...]
    bk = jnp.where(bk >= 0, bk, 0.01 * bk)
    Y = fc + bk
    y_ref[0] = Y[:, :HALF]
    y_ref[1] = Y[:, HALF:]
    nrm = jnp.sqrt(jnp.sum(Y * Y, axis=1, keepdims=True))
    yn_ref[...] = Y / jnp.maximum(nrm, 1e-12)


_dense = pl.pallas_call(
    _dense_body,
    grid=(NN // BN,),
    in_specs=[
        pl.BlockSpec((NC, BN, HALF), lambda i: (0, i, 0)),
        pl.BlockSpec((NC, BN, HALF), lambda i: (0, i, 0)),
        pl.BlockSpec((D, D), lambda i: (0, 0)),
        pl.BlockSpec((1, D), lambda i: (0, 0)),
        pl.BlockSpec((D, D), lambda i: (0, 0)),
        pl.BlockSpec((1, D), lambda i: (0, 0)),
    ],
    out_specs=[
        pl.BlockSpec((NC, BN, HALF), lambda i: (0, i, 0)),
        pl.BlockSpec((BN, D), lambda i: (i, 0)),
    ],
    out_shape=[
        jax.ShapeDtypeStruct((NC, NN, HALF), jnp.float32),
        jax.ShapeDtypeStruct((NN, D), jnp.float32),
    ],
)


def kernel(edge_index, edge_weight, user_emb, item_emb,
           fw0, fb0, fw1, fb1, fw2, fb2,
           bw0, bb0, bw1, bb1, bw2, bb2):
    pad = EPAD - EDG
    src = jnp.concatenate([edge_index[0].astype(jnp.int32),
                           jnp.zeros((pad,), jnp.int32)])
    dst = jnp.concatenate([edge_index[1].astype(jnp.int32),
                           jnp.zeros((pad,), jnp.int32)])
    w = jnp.concatenate([edge_weight.astype(jnp.float32),
                         jnp.zeros((pad,), jnp.float32)])
    wbits = lax.bitcast_convert_type(w, jnp.int32)
    edat = jnp.stack([src.reshape(-1, CHUNK), dst.reshape(-1, CHUNK),
                      wbits.reshape(-1, CHUNK)], axis=0)
    E0 = jnp.concatenate([user_emb, item_emb], axis=0)
    x3 = jnp.stack([E0[:, :HALF], E0[:, HALF:]], axis=0)
    layers = [(fw0, fb0, bw0, bb0), (fw1, fb1, bw1, bb1), (fw2, fb2, bw2, bb2)]
    outs = [E0]
    for (fw, fb, bw, bb) in layers:
        front3 = _spmm(x3, edat)
        y3, yn = _dense(front3, x3,
                        fw, fb.reshape(1, D), bw, bb.reshape(1, D))
        x3 = y3
        outs.append(yn)
    all_emb = jnp.concatenate(outs, axis=1)
    return all_emb[:N_U], all_emb[N_U:]
